# baseline (speedup 1.0000x reference)
import functools

import jax
import jax.numpy as jnp
from jax import lax
from jax.experimental import pallas as pl
from jax.experimental.pallas import tpu as pltpu
from jax.experimental.pallas import tpu_sc as plsc


def _sc_gather(user_table, item_table, user_idx, item_idx):
    info = plsc.get_sparse_core_info()
    NW = info.num_cores * info.num_subcores
    B = user_idx.shape[0]
    D = user_table.shape[1]
    b_per_w = B // NW
    CH = min(128, b_per_w)
    nch = b_per_w // CH
    mesh = plsc.VectorSubcoreMesh(core_axis_name="c", subcore_axis_name="s")

    @functools.partial(
        pl.kernel,
        mesh=mesh,
        out_type=[
            jax.ShapeDtypeStruct((B, D), jnp.float32),
            jax.ShapeDtypeStruct((B, D), jnp.float32),
        ],
        scratch_types=[
            pltpu.VMEM((b_per_w,), jnp.int32),
            pltpu.VMEM((b_per_w,), jnp.int32),
            pltpu.VMEM((2, CH, D), jnp.float32),
            pltpu.VMEM((2, CH, D), jnp.float32),
            [pltpu.SemaphoreType.DMA] * 2,
            [pltpu.SemaphoreType.DMA] * 2,
            [pltpu.SemaphoreType.DMA] * 2,
            [pltpu.SemaphoreType.DMA] * 2,
        ],
    )
    def k(ut_hbm, it_hbm, ui_hbm, ii_hbm, ue_hbm, ie_hbm,
          ui_v, ii_v, ubuf, ibuf, ugsem, igsem, uwsem, iwsem):
        wid = lax.axis_index("s") * info.num_cores + lax.axis_index("c")
        base = wid * b_per_w
        pltpu.sync_copy(ui_hbm.at[pl.ds(base, b_per_w)], ui_v)
        pltpu.sync_copy(ii_hbm.at[pl.ds(base, b_per_w)], ii_v)

        def gather(c, s):
            off = c * CH
            ug = pltpu.async_copy(
                ut_hbm.at[ui_v.at[pl.ds(off, CH)]], ubuf.at[s], ugsem[s])
            ig = pltpu.async_copy(
                it_hbm.at[ii_v.at[pl.ds(off, CH)]], ibuf.at[s], igsem[s])
            return ug, ig

        def writeback(c, s):
            off = c * CH
            uw = pltpu.async_copy(
                ubuf.at[s], ue_hbm.at[pl.ds(base + off, CH)], uwsem[s])
            iw = pltpu.async_copy(
                ibuf.at[s], ie_hbm.at[pl.ds(base + off, CH)], iwsem[s])
            return uw, iw

        g = gather(0, 0)
        w_prev = None
        for c in range(nch):
            s = c % 2
            if c + 1 < nch:
                if w_prev is not None:
                    w_prev[0].wait()
                    w_prev[1].wait()
                g_next = gather(c + 1, (c + 1) % 2)
            g[0].wait()
            g[1].wait()
            w = writeback(c, s)
            if c + 1 < nch:
                g = g_next
            w_prev = w
        w_prev[0].wait()
        w_prev[1].wait()

    return k(user_table, item_table, user_idx, item_idx)


def _mlp_body(ue_ref, ie_ref, w1u_ref, w1v_ref, b1_ref, w2_ref, b2_ref,
              w3t_ref, b3_ref, out_ref):
    h = jnp.dot(ue_ref[...], w1u_ref[...], preferred_element_type=jnp.float32)
    h = h + jnp.dot(ie_ref[...], w1v_ref[...], preferred_element_type=jnp.float32)
    h = jnp.maximum(h + b1_ref[...], 0.0)
    h = jnp.dot(h, w2_ref[...], preferred_element_type=jnp.float32)
    h = jnp.maximum(h + b2_ref[...], 0.0)
    ht = jnp.transpose(h)
    o = jnp.dot(w3t_ref[...], ht, preferred_element_type=jnp.float32) + b3_ref[...]
    out_ref[...] = jax.nn.sigmoid(o[0, :]) * 4.0 + 1.0


def _tc_mlp(ue, ie, W1u, W1v, b1, W2, b2, W3, b3, tb=8192):
    B, D = ue.shape
    H1 = W1u.shape[1]
    H2 = W2.shape[1]
    HP = 128
    W2p = jnp.pad(W2, ((0, 0), (0, HP - H2)))
    b2p = jnp.pad(b2, (0, HP - H2)).reshape(1, HP)
    W3t = jnp.pad(W3.T, ((0, 0), (0, HP - H2)))
    return pl.pallas_call(
        _mlp_body,
        grid=(B // tb,),
        in_specs=[
            pl.BlockSpec((tb, D), lambda i: (i, 0)),
            pl.BlockSpec((tb, D), lambda i: (i, 0)),
            pl.BlockSpec((D, H1), lambda i: (0, 0)),
            pl.BlockSpec((D, H1), lambda i: (0, 0)),
            pl.BlockSpec((1, H1), lambda i: (0, 0)),
            pl.BlockSpec((H1, HP), lambda i: (0, 0)),
            pl.BlockSpec((1, HP), lambda i: (0, 0)),
            pl.BlockSpec((1, HP), lambda i: (0, 0)),
            pl.BlockSpec((1, 1), lambda i: (0, 0)),
        ],
        out_specs=pl.BlockSpec((tb,), lambda i: (i,)),
        out_shape=jax.ShapeDtypeStruct((B,), jnp.float32),
    )(ue, ie, W1u, W1v, b1.reshape(1, H1), W2p, b2p, W3t,
      b3.reshape(1, 1))


def kernel(x, user_table, item_table, W1, b1, W2, b2, W3, b3):
    D = user_table.shape[1]
    B = x.shape[0]
    user_idx = x[:, 0].astype(jnp.int32)
    item_idx = x[:, 1].astype(jnp.int32)
    nchunk = 1
    bc = B // nchunk
    outs = []
    for c in range(nchunk):
        sl = slice(c * bc, (c + 1) * bc)
        ue, ie = _sc_gather(user_table, item_table, user_idx[sl], item_idx[sl])
        outs.append(_tc_mlp(ue, ie, W1[:D], W1[D:], b1, W2, b2, W3, b3))
    return jnp.concatenate(outs).reshape(-1, 1)


# speedup vs baseline: 1.0236x; 1.0236x over previous
import functools

import jax
import jax.numpy as jnp
from jax import lax
from jax.experimental import pallas as pl
from jax.experimental.pallas import tpu as pltpu
from jax.experimental.pallas import tpu_sc as plsc


def _sc_gather(user_table, item_table, user_idx, item_idx):
    info = plsc.get_sparse_core_info()
    NW = info.num_cores * info.num_subcores
    B = user_idx.shape[0]
    D = user_table.shape[1]
    b_per_w = B // NW
    CH = min(128, b_per_w)
    nch = b_per_w // CH
    mesh = plsc.VectorSubcoreMesh(core_axis_name="c", subcore_axis_name="s")

    @functools.partial(
        pl.kernel,
        mesh=mesh,
        out_type=[
            jax.ShapeDtypeStruct((B, D), jnp.float32),
            jax.ShapeDtypeStruct((B, D), jnp.float32),
        ],
        scratch_types=[
            pltpu.VMEM((b_per_w,), jnp.int32),
            pltpu.VMEM((b_per_w,), jnp.int32),
            pltpu.VMEM((2, CH, D), jnp.float32),
            pltpu.VMEM((2, CH, D), jnp.float32),
            [pltpu.SemaphoreType.DMA] * 2,
            [pltpu.SemaphoreType.DMA] * 2,
            [pltpu.SemaphoreType.DMA] * 2,
            [pltpu.SemaphoreType.DMA] * 2,
        ],
    )
    def k(ut_hbm, it_hbm, ui_hbm, ii_hbm, ue_hbm, ie_hbm,
          ui_v, ii_v, ubuf, ibuf, ugsem, igsem, uwsem, iwsem):
        wid = lax.axis_index("s") * info.num_cores + lax.axis_index("c")
        base = wid * b_per_w
        pltpu.sync_copy(ui_hbm.at[pl.ds(base, b_per_w)], ui_v)
        pltpu.sync_copy(ii_hbm.at[pl.ds(base, b_per_w)], ii_v)

        def gather(c, s):
            off = c * CH
            ug = pltpu.async_copy(
                ut_hbm.at[ui_v.at[pl.ds(off, CH)]], ubuf.at[s], ugsem[s])
            ig = pltpu.async_copy(
                it_hbm.at[ii_v.at[pl.ds(off, CH)]], ibuf.at[s], igsem[s])
            return ug, ig

        def writeback(c, s):
            off = c * CH
            uw = pltpu.async_copy(
                ubuf.at[s], ue_hbm.at[pl.ds(base + off, CH)], uwsem[s])
            iw = pltpu.async_copy(
                ibuf.at[s], ie_hbm.at[pl.ds(base + off, CH)], iwsem[s])
            return uw, iw

        g = gather(0, 0)
        w_prev = None
        for c in range(nch):
            s = c % 2
            if c + 1 < nch:
                if w_prev is not None:
                    w_prev[0].wait()
                    w_prev[1].wait()
                g_next = gather(c + 1, (c + 1) % 2)
            g[0].wait()
            g[1].wait()
            w = writeback(c, s)
            if c + 1 < nch:
                g = g_next
            w_prev = w
        w_prev[0].wait()
        w_prev[1].wait()

    return k(user_table, item_table, user_idx, item_idx)


def _mlp_body(ue_ref, ie_ref, w1u_ref, w1v_ref, b1_ref, w2_ref, b2_ref,
              w3t_ref, b3_ref, out_ref):
    h = jnp.dot(ue_ref[...], w1u_ref[...], preferred_element_type=jnp.float32)
    h = h + jnp.dot(ie_ref[...], w1v_ref[...], preferred_element_type=jnp.float32)
    h = jnp.maximum(h + b1_ref[...], 0.0)
    h = jnp.dot(h, w2_ref[...], preferred_element_type=jnp.float32)
    h = jnp.maximum(h + b2_ref[...], 0.0)
    ht = jnp.transpose(h)
    o = jnp.dot(w3t_ref[...], ht, preferred_element_type=jnp.float32) + b3_ref[...]
    out_ref[...] = jax.nn.sigmoid(o[0, :]) * 4.0 + 1.0


def _tc_mlp(ue, ie, W1u, W1v, b1, W2, b2, W3, b3, tb=4096):
    B, D = ue.shape
    H1 = W1u.shape[1]
    H2 = W2.shape[1]
    HP = 128
    W2p = jnp.pad(W2, ((0, 0), (0, HP - H2)))
    b2p = jnp.pad(b2, (0, HP - H2)).reshape(1, HP)
    W3t = jnp.pad(W3.T, ((0, 0), (0, HP - H2)))
    return pl.pallas_call(
        _mlp_body,
        grid=(B // tb,),
        in_specs=[
            pl.BlockSpec((tb, D), lambda i: (i, 0)),
            pl.BlockSpec((tb, D), lambda i: (i, 0)),
            pl.BlockSpec((D, H1), lambda i: (0, 0)),
            pl.BlockSpec((D, H1), lambda i: (0, 0)),
            pl.BlockSpec((1, H1), lambda i: (0, 0)),
            pl.BlockSpec((H1, HP), lambda i: (0, 0)),
            pl.BlockSpec((1, HP), lambda i: (0, 0)),
            pl.BlockSpec((1, HP), lambda i: (0, 0)),
            pl.BlockSpec((1, 1), lambda i: (0, 0)),
        ],
        out_specs=pl.BlockSpec((tb,), lambda i: (i,)),
        out_shape=jax.ShapeDtypeStruct((B,), jnp.float32),
    )(ue, ie, W1u, W1v, b1.reshape(1, H1), W2p, b2p, W3t,
      b3.reshape(1, 1))


def kernel(x, user_table, item_table, W1, b1, W2, b2, W3, b3):
    D = user_table.shape[1]
    B = x.shape[0]
    user_idx = x[:, 0].astype(jnp.int32)
    item_idx = x[:, 1].astype(jnp.int32)
    nchunk = 1
    bc = B // nchunk
    outs = []
    for c in range(nchunk):
        sl = slice(c * bc, (c + 1) * bc)
        ue, ie = _sc_gather(user_table, item_table, user_idx[sl], item_idx[sl])
        outs.append(_tc_mlp(ue, ie, W1[:D], W1[D:], b1, W2, b2, W3, b3))
    return jnp.concatenate(outs).reshape(-1, 1)
